# baseline (speedup 1.0000x reference)
import functools
import math

import jax
import jax.numpy as jnp
from jax import lax
from jax.experimental import pallas as pl
from jax.experimental.pallas import tpu as pltpu
from jax.experimental.pallas import tpu_sc as plsc

_CHUNK = 100
_NBUF = 2


@functools.lru_cache(maxsize=None)
def _build(total_chunks, chunk, seq_chunks, embed, scale):
    info = plsc.get_sparse_core_info()
    nc, ns = info.num_cores, info.num_subcores
    nw = nc * ns
    assert total_chunks % nw == 0
    cpw = total_chunks // nw
    assert seq_chunks == 2 and _NBUF % 2 == 0 and cpw % _NBUF == 0
    assert cpw >= 2 * _NBUF
    seq_len = chunk * seq_chunks
    n_j = embed // 16
    assert embed % 16 == 0

    mesh = plsc.VectorSubcoreMesh(core_axis_name="c", subcore_axis_name="s")

    @functools.partial(
        pl.kernel,
        out_type=jax.ShapeDtypeStruct((total_chunks, chunk, embed), jnp.float32),
        mesh=mesh,
        compiler_params=pltpu.CompilerParams(use_tc_tiling_on_sc=False),
        scratch_types=[
            pltpu.VMEM((cpw, chunk), jnp.int32),
            pltpu.VMEM((seq_len, embed), jnp.float32),
            [pltpu.VMEM((chunk, embed), jnp.float32) for _ in range(_NBUF)],
            [pltpu.VMEM((chunk, embed), jnp.float32) for _ in range(_NBUF)],
            [pltpu.SemaphoreType.DMA for _ in range(_NBUF)],
            [pltpu.SemaphoreType.DMA for _ in range(_NBUF)],
        ],
    )
    def emb_kernel(ids_hbm, tok_hbm, pos_hbm, out_hbm,
                   idx_v, pos_v, gbufs, obufs, gsems, ssems):
        wid = lax.axis_index("s") * nc + lax.axis_index("c")
        cbase = wid * cpw

        pltpu.sync_copy(ids_hbm.at[pl.ds(cbase, cpw)], idx_v)
        pltpu.sync_copy(pos_hbm.at[pl.ds(0, seq_len)], pos_v)

        def start_gather(b, k):
            pltpu.async_copy(tok_hbm.at[idx_v.at[k]], gbufs[b], gsems[b])

        def wait_gather(b, k):
            pltpu.make_async_copy(tok_hbm.at[idx_v.at[k]], gbufs[b],
                                  gsems[b]).wait()

        def start_store(b, k):
            pltpu.async_copy(obufs[b], out_hbm.at[cbase + k], ssems[b])

        def wait_store(b, k):
            pltpu.make_async_copy(obufs[b], out_hbm.at[cbase + k],
                                  ssems[b]).wait()

        def compute(b):
            poff = (b % 2) * chunk

            @pl.loop(0, chunk)
            def _(i):
                for j in range(n_j):
                    sl = pl.ds(j * 16, 16)
                    obufs[b][i, sl] = gbufs[b][i, sl] * scale + pos_v[poff + i, sl]

        for b in range(_NBUF):
            start_gather(b, b)
        for b in range(_NBUF):
            wait_gather(b, b)
            compute(b)
            start_gather(b, b + _NBUF)
            start_store(b, b)

        @pl.loop(_NBUF, cpw - _NBUF, step=_NBUF)
        def _(t):
            for b in range(_NBUF):
                k = t + b
                wait_gather(b, k)
                wait_store(b, k - _NBUF)
                compute(b)
                start_gather(b, k + _NBUF)
                start_store(b, k)

        for b in range(_NBUF):
            k = cpw - _NBUF + b
            wait_gather(b, k)
            wait_store(b, k - _NBUF)
            compute(b)
            start_store(b, k)
        for b in range(_NBUF):
            wait_store(b, cpw - _NBUF + b)

    return emb_kernel


def kernel(input_ids, key_padding_mask, token_table, pos_table):
    del key_padding_mask
    bsz, seq = input_ids.shape
    _, embed = token_table.shape
    assert seq % _CHUNK == 0
    total_chunks = (bsz * seq) // _CHUNK
    ids2 = input_ids.astype(jnp.int32).reshape(total_chunks, _CHUNK)
    fn = _build(total_chunks, _CHUNK, seq // _CHUNK, embed, math.sqrt(embed))
    out = fn(ids2, token_table, pos_table)
    return out.reshape(bsz, seq, embed)
